# Initial kernel scaffold; baseline (speedup 1.0000x reference)
import functools

import jax
import jax.numpy as jnp
from jax import lax
from jax.experimental import pallas as pl
from jax.experimental.pallas import tpu as pltpu
from jax.experimental.pallas import tpu_sc as plsc

HIDDEN = 32
NC = 2
NS = 16
NW = NC * NS
B_TOTAL = 4096 * 200
BPW = B_TOTAL // NW
CHUNK = 128
NCHUNK = BPW // CHUNK

_mesh = plsc.VectorSubcoreMesh(core_axis_name="c", subcore_axis_name="s")


@functools.partial(
    pl.kernel,
    mesh=_mesh,
    out_type=jax.ShapeDtypeStruct((NW, NCHUNK, CHUNK, HIDDEN), jnp.float32),
    scratch_types=[
        pltpu.VMEM((NCHUNK, CHUNK), jnp.int32),
        pltpu.VMEM((CHUNK, HIDDEN), jnp.float32),
        pltpu.SemaphoreType.DMA,
    ],
)
def _gather_kernel(idx_hbm, table_hbm, out_hbm, idx_v, rows_v, sem):
    wid = lax.axis_index("s") * NC + lax.axis_index("c")
    pltpu.sync_copy(idx_hbm.at[wid], idx_v)

    def body(j, carry):
        pltpu.async_copy(table_hbm.at[idx_v.at[j]], rows_v, sem).wait()
        pltpu.sync_copy(rows_v, out_hbm.at[wid, j])
        return carry

    lax.fori_loop(0, NCHUNK, body, 0)


def kernel(obs, obs_embedding_weight):
    idx = obs.reshape(NW, NCHUNK, CHUNK).astype(jnp.int32)
    out = _gather_kernel(idx, obs_embedding_weight)
    return out.reshape(4096, 200, HIDDEN)


# speedup vs baseline: 1.3073x; 1.3073x over previous
import functools

import jax
import jax.numpy as jnp
from jax import lax
from jax.experimental import pallas as pl
from jax.experimental.pallas import tpu as pltpu
from jax.experimental.pallas import tpu_sc as plsc

HIDDEN = 32
NC = 2
NS = 16
NW = NC * NS
B_TOTAL = 4096 * 200
BPW = B_TOTAL // NW
CHUNK = 128
NCHUNK = BPW // CHUNK

_mesh = plsc.VectorSubcoreMesh(core_axis_name="c", subcore_axis_name="s")


@functools.partial(
    pl.kernel,
    mesh=_mesh,
    compiler_params=pltpu.CompilerParams(use_tc_tiling_on_sc=False),
    out_type=jax.ShapeDtypeStruct((NW, NCHUNK, CHUNK, HIDDEN), jnp.float32),
    scratch_types=[
        pltpu.VMEM((NCHUNK, CHUNK), jnp.int32),
        pltpu.VMEM((CHUNK, HIDDEN), jnp.float32),
        pltpu.SemaphoreType.DMA,
    ],
)
def _gather_kernel(idx_hbm, table_hbm, out_hbm, idx_v, rows_v, sem):
    wid = lax.axis_index("s") * NC + lax.axis_index("c")
    pltpu.sync_copy(idx_hbm.at[wid], idx_v)

    def body(j, carry):
        pltpu.async_copy(table_hbm.at[idx_v.at[j]], rows_v, sem).wait()
        pltpu.sync_copy(rows_v, out_hbm.at[wid, j])
        return carry

    lax.fori_loop(0, NCHUNK, body, 0)


def kernel(obs, obs_embedding_weight):
    idx = obs.reshape(NW, NCHUNK, CHUNK).astype(jnp.int32)
    out = _gather_kernel(idx, obs_embedding_weight)
    return out.reshape(4096, 200, HIDDEN)


# speedup vs baseline: 1.4974x; 1.1454x over previous
import functools

import jax
import jax.numpy as jnp
from jax import lax
from jax.experimental import pallas as pl
from jax.experimental.pallas import tpu as pltpu
from jax.experimental.pallas import tpu_sc as plsc

HIDDEN = 32
NC = 2
NS = 16
NW = NC * NS
B_TOTAL = 4096 * 200
BPW = B_TOTAL // NW
CHUNK = 128
K = 8
BLOCK = K * CHUNK
NCHUNK = BPW // CHUNK
NBLK = BPW // BLOCK

_mesh = plsc.VectorSubcoreMesh(core_axis_name="c", subcore_axis_name="s")


@functools.partial(
    pl.kernel,
    mesh=_mesh,
    compiler_params=pltpu.CompilerParams(use_tc_tiling_on_sc=False),
    out_type=jax.ShapeDtypeStruct((NW, NBLK, K, CHUNK, HIDDEN), jnp.float32),
    scratch_types=[
        pltpu.VMEM((NCHUNK, CHUNK), jnp.int32),
        pltpu.VMEM((2, K, CHUNK, HIDDEN), jnp.float32),
        pltpu.SemaphoreType.DMA,
        pltpu.SemaphoreType.DMA,
    ],
)
def _gather_kernel(idx_hbm, table_hbm, out_hbm, idx_v, rows_v, gsem, osem):
    wid = lax.axis_index("s") * NC + lax.axis_index("c")
    pltpu.sync_copy(idx_hbm.at[wid], idx_v)

    for k in range(K):
        pltpu.async_copy(table_hbm.at[idx_v.at[k]], rows_v.at[0, k], gsem)

    @pl.loop(0, NBLK)
    def _(i):
        g = lax.rem(i, 2)
        pltpu.make_async_copy(out_hbm.at[wid, 0], rows_v.at[g], gsem).wait()

        @pl.when(i > 0)
        def _():
            pltpu.make_async_copy(out_hbm.at[wid, 0], rows_v.at[g], osem).wait()

        @pl.when(i + 1 < NBLK)
        def _():
            for k in range(K):
                pltpu.async_copy(
                    table_hbm.at[idx_v.at[(i + 1) * K + k]],
                    rows_v.at[1 - g, k],
                    gsem,
                )

        pltpu.async_copy(rows_v.at[g], out_hbm.at[wid, i], osem)

    pltpu.make_async_copy(out_hbm.at[wid, 0], rows_v.at[0], osem).wait()


def kernel(obs, obs_embedding_weight):
    idx = obs.reshape(NW, NCHUNK, CHUNK).astype(jnp.int32)
    out = _gather_kernel(idx, obs_embedding_weight)
    return out.reshape(4096, 200, HIDDEN)
